# baseline (speedup 1.0000x reference)
import functools

import jax
import jax.numpy as jnp
from jax import lax
from jax.experimental import pallas as pl
from jax.experimental.pallas import tpu as pltpu
from jax.experimental.pallas import tpu_sc as plsc

N_NODES = 10000
N_EDGES = 320000
D_FEAT = 128

NS = 16
L = 16
E_PER_W = N_EDGES // NS
N_PAD = 10240
NODES_W = N_PAD // NS
CHUNK = 128
N_CHUNK = NODES_W // CHUNK
ROWS_LAST = N_NODES - (NS - 1) * NODES_W

_mesh = plsc.VectorSubcoreMesh(
    core_axis_name="c", subcore_axis_name="s", num_cores=1, num_subcores=NS
)

_params = pltpu.CompilerParams(
    needs_layout_passes=False, use_tc_tiling_on_sc=False
)

_UNROLL = 5


@functools.partial(
    pl.kernel,
    out_type=(
        jax.ShapeDtypeStruct((NS, N_PAD), jnp.float32),
        jax.ShapeDtypeStruct((N_NODES, D_FEAT), jnp.float32),
    ),
    mesh=_mesh,
    scratch_types=[
        pltpu.VMEM((E_PER_W,), jnp.int32),
        pltpu.VMEM((N_PAD,), jnp.float32),
        pltpu.VMEM((NS, NODES_W), jnp.float32),
        pltpu.VMEM((N_CHUNK, CHUNK), jnp.int32),
        pltpu.VMEM((NODES_W, D_FEAT), jnp.float32),
        pltpu.SemaphoreType.DMA,
        pltpu.SemaphoreType.DMA,
    ],
    compiler_params=_params,
)
def _membership(idx_hbm, table_hbm, part_hbm, out_hbm,
                idx_v, flags_v, pblk_v, rowsel_v, out_v, sem, gsem):
    sid = lax.axis_index("s")

    ebase = sid * E_PER_W
    cp = pltpu.async_copy(idx_hbm.at[pl.ds(ebase, E_PER_W)], idx_v, sem)

    zero = jnp.zeros((L,), jnp.float32)

    def zbody(i, carry):
        for k in range(_UNROLL):
            flags_v[pl.ds((i * _UNROLL + k) * L, L)] = zero
        return carry

    lax.fori_loop(0, N_PAD // (L * _UNROLL), zbody, 0)
    cp.wait()

    one = jnp.ones((L,), jnp.float32)

    def sbody(i, carry):
        for k in range(_UNROLL):
            iv = idx_v[pl.ds((i * _UNROLL + k) * L, L)]
            plsc.store_scatter(flags_v, [iv], one)
        return carry

    lax.fori_loop(0, E_PER_W // (L * _UNROLL), sbody, 0)
    pltpu.sync_copy(flags_v, part_hbm.at[sid])

    plsc.subcore_barrier()

    nbase = sid * NODES_W
    pltpu.sync_copy(part_hbm.at[:, pl.ds(nbase, NODES_W)], pblk_v)

    for g in range(NODES_W // L):
        acc = pblk_v[0, pl.ds(g * L, L)]
        for r in range(1, NS):
            acc = acc + pblk_v[r, pl.ds(g * L, L)]
        sel = jnp.where(acc > 0.0, 1, 0).astype(jnp.int32)
        rowsel_v[g * L // CHUNK, pl.ds((g * L) % CHUNK, L)] = sel

    copies = [
        pltpu.async_copy(
            table_hbm.at[rowsel_v.at[j]],
            out_v.at[pl.ds(j * CHUNK, CHUNK)],
            gsem,
        )
        for j in range(N_CHUNK)
    ]
    for c in copies:
        c.wait()

    @pl.when(sid < NS - 1)
    def _():
        pltpu.sync_copy(
            out_v.at[pl.ds(0, NODES_W)],
            out_hbm.at[pl.ds(nbase, NODES_W)],
        )

    @pl.when(sid == NS - 1)
    def _():
        pltpu.sync_copy(
            out_v.at[pl.ds(0, ROWS_LAST)],
            out_hbm.at[pl.ds(nbase, ROWS_LAST)],
        )


def kernel(source_node_representation_with_coefficient, edge_index):
    del source_node_representation_with_coefficient
    idx = edge_index[1]
    table = jnp.concatenate(
        [jnp.zeros((1, D_FEAT), jnp.float32), jnp.ones((1, D_FEAT), jnp.float32)]
    )
    _, out = _membership(idx, table)
    return out


# speedup vs baseline: 8.4780x; 8.4780x over previous
import functools

import jax
import jax.numpy as jnp
from jax import lax
from jax.experimental import pallas as pl
from jax.experimental.pallas import tpu as pltpu
from jax.experimental.pallas import tpu_sc as plsc

N_NODES = 10000
N_EDGES = 320000
D_FEAT = 128

NS = 16
L = 16
E_PER_W = N_EDGES // NS
N_PAD = 10240
NODES_W = N_PAD // NS
CHUNK = 128
N_CHUNK = NODES_W // CHUNK
ROWS_LAST = N_NODES - (NS - 1) * NODES_W

_mesh = plsc.VectorSubcoreMesh(
    core_axis_name="c", subcore_axis_name="s", num_cores=1, num_subcores=NS
)

_params = pltpu.CompilerParams(
    needs_layout_passes=False, use_tc_tiling_on_sc=False
)

_UNROLL = 5


@functools.partial(
    pl.kernel,
    out_type=(
        jax.ShapeDtypeStruct((NS, N_PAD), jnp.float32),
        jax.ShapeDtypeStruct((N_NODES, D_FEAT), jnp.float32),
    ),
    mesh=_mesh,
    scratch_types=[
        pltpu.VMEM((E_PER_W,), jnp.int32),
        pltpu.VMEM((N_PAD,), jnp.float32),
        pltpu.VMEM((NS, NODES_W), jnp.float32),
        pltpu.VMEM((NODES_W,), jnp.float32),
        pltpu.VMEM((NODES_W, D_FEAT), jnp.float32),
        pltpu.SemaphoreType.DMA,
    ],
    compiler_params=_params,
)
def _membership(idx_hbm, part_hbm, out_hbm,
                idx_v, flags_v, pblk_v, nflags_v, out_v, sem):
    sid = lax.axis_index("s")

    ebase = sid * E_PER_W
    cp = pltpu.async_copy(idx_hbm.at[pl.ds(ebase, E_PER_W)], idx_v, sem)

    zero = jnp.zeros((L,), jnp.float32)

    def zbody(i, carry):
        for k in range(_UNROLL):
            flags_v[pl.ds((i * _UNROLL + k) * L, L)] = zero
        return carry

    lax.fori_loop(0, N_PAD // (L * _UNROLL), zbody, 0)
    cp.wait()

    one = jnp.ones((L,), jnp.float32)

    def sbody(i, carry):
        for k in range(_UNROLL):
            iv = idx_v[pl.ds((i * _UNROLL + k) * L, L)]
            plsc.store_scatter(flags_v, [iv], one)
        return carry

    lax.fori_loop(0, E_PER_W // (L * _UNROLL), sbody, 0)
    pltpu.sync_copy(flags_v, part_hbm.at[sid])

    plsc.subcore_barrier()

    nbase = sid * NODES_W
    pltpu.sync_copy(part_hbm.at[:, pl.ds(nbase, NODES_W)], pblk_v)

    for g in range(NODES_W // L):
        acc = pblk_v[0, pl.ds(g * L, L)]
        for r in range(1, NS):
            acc = acc + pblk_v[r, pl.ds(g * L, L)]
        nflags_v[pl.ds(g * L, L)] = jnp.where(acc > 0.0, 1.0, 0.0)

    def bbody(n, carry):
        iv = jnp.full((L,), n, dtype=jnp.int32)
        fv = plsc.load_gather(nflags_v, [iv])
        for k in range(D_FEAT // L):
            out_v[n, pl.ds(k * L, L)] = fv
        return carry

    lax.fori_loop(0, NODES_W, bbody, 0)

    @pl.when(sid < NS - 1)
    def _():
        pltpu.sync_copy(
            out_v.at[pl.ds(0, NODES_W)],
            out_hbm.at[pl.ds(nbase, NODES_W)],
        )

    @pl.when(sid == NS - 1)
    def _():
        pltpu.sync_copy(
            out_v.at[pl.ds(0, ROWS_LAST)],
            out_hbm.at[pl.ds(nbase, ROWS_LAST)],
        )


def kernel(source_node_representation_with_coefficient, edge_index):
    del source_node_representation_with_coefficient
    idx = edge_index[1]
    _, out = _membership(idx)
    return out


# speedup vs baseline: 9.1216x; 1.0759x over previous
import functools

import jax
import jax.numpy as jnp
from jax import lax
from jax.experimental import pallas as pl
from jax.experimental.pallas import tpu as pltpu
from jax.experimental.pallas import tpu_sc as plsc

N_NODES = 10000
N_EDGES = 320000
D_FEAT = 128

NC = 2
NS = 16
L = 16
NW = NC * NS
E_PER_T = N_EDGES // NS
N_PAD = 10240
NODES_W = N_PAD // NW
ROWS_LAST = N_NODES - (NW - 1) * NODES_W

_mesh = plsc.VectorSubcoreMesh(
    core_axis_name="c", subcore_axis_name="s", num_cores=NC, num_subcores=NS
)

_params = pltpu.CompilerParams(
    needs_layout_passes=False, use_tc_tiling_on_sc=False
)

_UNROLL = 5


@functools.partial(
    pl.kernel,
    out_type=(
        jax.ShapeDtypeStruct((NC, NS, N_PAD), jnp.float32),
        jax.ShapeDtypeStruct((N_NODES, D_FEAT), jnp.float32),
    ),
    mesh=_mesh,
    scratch_types=[
        pltpu.VMEM((E_PER_T,), jnp.int32),
        pltpu.VMEM((N_PAD,), jnp.float32),
        pltpu.VMEM((NS, NODES_W), jnp.float32),
        pltpu.VMEM((NODES_W,), jnp.float32),
        pltpu.VMEM((NODES_W, D_FEAT), jnp.float32),
        pltpu.SemaphoreType.DMA,
    ],
    compiler_params=_params,
)
def _membership(idx_hbm, part_hbm, out_hbm,
                idx_v, flags_v, pblk_v, nflags_v, out_v, sem):
    cid = lax.axis_index("c")
    sid = lax.axis_index("s")

    ebase = sid * E_PER_T
    cp = pltpu.async_copy(idx_hbm.at[pl.ds(ebase, E_PER_T)], idx_v, sem)

    zero = jnp.zeros((L,), jnp.float32)

    def zbody(i, carry):
        for k in range(_UNROLL):
            flags_v[pl.ds((i * _UNROLL + k) * L, L)] = zero
        return carry

    lax.fori_loop(0, N_PAD // (L * _UNROLL), zbody, 0)
    cp.wait()

    one = jnp.ones((L,), jnp.float32)

    def sbody(i, carry):
        for k in range(_UNROLL):
            iv = idx_v[pl.ds((i * _UNROLL + k) * L, L)]
            plsc.store_scatter(flags_v, [iv], one)
        return carry

    lax.fori_loop(0, E_PER_T // (L * _UNROLL), sbody, 0)
    pltpu.sync_copy(flags_v, part_hbm.at[cid, sid])

    plsc.subcore_barrier()

    wid = cid * NS + sid
    nbase = wid * NODES_W
    pltpu.sync_copy(part_hbm.at[cid, :, pl.ds(nbase, NODES_W)], pblk_v)

    for g in range(NODES_W // L):
        acc = pblk_v[0, pl.ds(g * L, L)]
        for r in range(1, NS):
            acc = acc + pblk_v[r, pl.ds(g * L, L)]
        nflags_v[pl.ds(g * L, L)] = jnp.where(acc > 0.0, 1.0, 0.0)

    def bbody(n, carry):
        iv = jnp.full((L,), n, dtype=jnp.int32)
        fv = plsc.load_gather(nflags_v, [iv])
        for k in range(D_FEAT // L):
            out_v[n, pl.ds(k * L, L)] = fv
        return carry

    lax.fori_loop(0, NODES_W, bbody, 0)

    @pl.when(wid < NW - 1)
    def _():
        pltpu.sync_copy(
            out_v.at[pl.ds(0, NODES_W)],
            out_hbm.at[pl.ds(nbase, NODES_W)],
        )

    @pl.when(wid == NW - 1)
    def _():
        pltpu.sync_copy(
            out_v.at[pl.ds(0, ROWS_LAST)],
            out_hbm.at[pl.ds(nbase, ROWS_LAST)],
        )


def kernel(source_node_representation_with_coefficient, edge_index):
    del source_node_representation_with_coefficient
    idx = edge_index[1]
    _, out = _membership(idx)
    return out


# speedup vs baseline: 9.4012x; 1.0307x over previous
import functools

import jax
import jax.numpy as jnp
from jax import lax
from jax.experimental import pallas as pl
from jax.experimental.pallas import tpu as pltpu
from jax.experimental.pallas import tpu_sc as plsc

N_NODES = 10000
N_EDGES = 320000
D_FEAT = 128

NC = 2
NS = 16
L = 16
NW = NC * NS
E_PER_T = N_EDGES // NW
N_PAD = 10240

_mesh = plsc.VectorSubcoreMesh(
    core_axis_name="c", subcore_axis_name="s", num_cores=NC, num_subcores=NS
)

_params = pltpu.CompilerParams(
    needs_layout_passes=False, use_tc_tiling_on_sc=False
)

_UNROLL = 5


@functools.partial(
    pl.kernel,
    out_type=jax.ShapeDtypeStruct((NW, N_PAD), jnp.float32),
    mesh=_mesh,
    scratch_types=[
        pltpu.VMEM((E_PER_T,), jnp.int32),
        pltpu.VMEM((N_PAD,), jnp.float32),
        pltpu.SemaphoreType.DMA,
    ],
    compiler_params=_params,
)
def _membership_scatter(idx_hbm, part_hbm, idx_v, flags_v, sem):
    wid = lax.axis_index("c") * NS + lax.axis_index("s")
    ebase = wid * E_PER_T
    cp = pltpu.async_copy(idx_hbm.at[pl.ds(ebase, E_PER_T)], idx_v, sem)

    zero = jnp.zeros((L,), jnp.float32)

    def zbody(i, carry):
        for k in range(_UNROLL):
            flags_v[pl.ds((i * _UNROLL + k) * L, L)] = zero
        return carry

    lax.fori_loop(0, N_PAD // (L * _UNROLL), zbody, 0)
    cp.wait()

    one = jnp.ones((L,), jnp.float32)

    def sbody(i, carry):
        for k in range(_UNROLL):
            iv = idx_v[pl.ds((i * _UNROLL + k) * L, L)]
            plsc.store_scatter(flags_v, [iv], one)
        return carry

    lax.fori_loop(0, E_PER_T // (L * _UNROLL), sbody, 0)
    pltpu.sync_copy(flags_v, part_hbm.at[wid])


B_TC = 2048


def _reduce_broadcast_tc(part_ref, out_ref):
    p = part_ref[...]
    ones = jnp.ones((NW, 1), jnp.float32)
    col = lax.dot_general(
        p, ones, (((0,), (0,)), ((), ())), preferred_element_type=jnp.float32
    )
    flag = jnp.where(col > 0.0, 1.0, 0.0)
    out_ref[...] = jnp.broadcast_to(flag, (B_TC, D_FEAT))


_reduce_broadcast = pl.pallas_call(
    _reduce_broadcast_tc,
    grid=(N_PAD // B_TC,),
    in_specs=[pl.BlockSpec((NW, B_TC), lambda i: (0, i))],
    out_specs=pl.BlockSpec((B_TC, D_FEAT), lambda i: (i, 0)),
    out_shape=jax.ShapeDtypeStruct((N_NODES, D_FEAT), jnp.float32),
)


def kernel(source_node_representation_with_coefficient, edge_index):
    del source_node_representation_with_coefficient
    idx = edge_index[1]
    part = _membership_scatter(idx)
    return _reduce_broadcast(part)


# speedup vs baseline: 12.1075x; 1.2879x over previous
import functools

import jax
import jax.numpy as jnp
from jax import lax
from jax.experimental import pallas as pl
from jax.experimental.pallas import tpu as pltpu
from jax.experimental.pallas import tpu_sc as plsc

N_NODES = 10000
N_EDGES = 320000
D_FEAT = 128

NC = 2
NS = 16
L = 16
NW = NC * NS
E_PER_T = N_EDGES // NW
N_PAD = 10240

_mesh = plsc.VectorSubcoreMesh(
    core_axis_name="c", subcore_axis_name="s", num_cores=NC, num_subcores=NS
)

_params = pltpu.CompilerParams(
    needs_layout_passes=False, use_tc_tiling_on_sc=False
)

_UNROLL = 5


@functools.partial(
    pl.kernel,
    out_type=jax.ShapeDtypeStruct((NW, N_PAD), jnp.float32),
    mesh=_mesh,
    scratch_types=[
        pltpu.VMEM((E_PER_T,), jnp.int32),
        pltpu.VMEM((N_PAD,), jnp.float32),
        pltpu.SemaphoreType.DMA,
    ],
    compiler_params=_params,
)
def _membership_scatter(ei_hbm, part_hbm, idx_v, flags_v, sem):
    wid = lax.axis_index("c") * NS + lax.axis_index("s")
    ebase = wid * E_PER_T
    cp = pltpu.async_copy(ei_hbm.at[1, pl.ds(ebase, E_PER_T)], idx_v, sem)

    zero = jnp.zeros((L,), jnp.float32)

    def zbody(i, carry):
        for k in range(_UNROLL):
            flags_v[pl.ds((i * _UNROLL + k) * L, L)] = zero
        return carry

    lax.fori_loop(0, N_PAD // (L * _UNROLL), zbody, 0)
    cp.wait()

    one = jnp.ones((L,), jnp.float32)

    def sbody(i, carry):
        for k in range(_UNROLL):
            iv = idx_v[pl.ds((i * _UNROLL + k) * L, L)]
            plsc.store_scatter(flags_v, [iv], one)
        return carry

    lax.fori_loop(0, E_PER_T // (L * _UNROLL), sbody, 0)
    pltpu.sync_copy(flags_v, part_hbm.at[wid])


B_TC = 2048


def _reduce_broadcast_tc(part_ref, out_ref):
    p = part_ref[...]
    ones = jnp.ones((NW, 1), jnp.float32)
    col = lax.dot_general(
        p, ones, (((0,), (0,)), ((), ())), preferred_element_type=jnp.float32
    )
    flag = jnp.where(col > 0.0, 1.0, 0.0)
    out_ref[...] = jnp.broadcast_to(flag, (B_TC, D_FEAT))


_reduce_broadcast = pl.pallas_call(
    _reduce_broadcast_tc,
    grid=(N_PAD // B_TC,),
    in_specs=[pl.BlockSpec((NW, B_TC), lambda i: (0, i))],
    out_specs=pl.BlockSpec((B_TC, D_FEAT), lambda i: (i, 0)),
    out_shape=jax.ShapeDtypeStruct((N_NODES, D_FEAT), jnp.float32),
)


def kernel(source_node_representation_with_coefficient, edge_index):
    del source_node_representation_with_coefficient
    part = _membership_scatter(edge_index)
    return _reduce_broadcast(part)


# speedup vs baseline: 12.5660x; 1.0379x over previous
import functools

import jax
import jax.numpy as jnp
from jax import lax
from jax.experimental import pallas as pl
from jax.experimental.pallas import tpu as pltpu
from jax.experimental.pallas import tpu_sc as plsc

N_NODES = 10000
N_EDGES = 320000
D_FEAT = 128

NC = 2
NS = 16
L = 16
NW = NC * NS
E_PER_T = N_EDGES // NW
N_PAD = 10240

_mesh = plsc.VectorSubcoreMesh(
    core_axis_name="c", subcore_axis_name="s", num_cores=NC, num_subcores=NS
)

_params = pltpu.CompilerParams(
    needs_layout_passes=False, use_tc_tiling_on_sc=False
)

_UNROLL = 5


@functools.partial(
    pl.kernel,
    out_type=jax.ShapeDtypeStruct((NW, N_PAD), jnp.float32),
    mesh=_mesh,
    scratch_types=[
        pltpu.VMEM((E_PER_T,), jnp.int32),
        pltpu.VMEM((N_PAD,), jnp.float32),
        pltpu.SemaphoreType.DMA,
    ],
    compiler_params=_params,
)
def _membership_scatter(ei_hbm, part_hbm, idx_v, flags_v, sem):
    wid = lax.axis_index("c") * NS + lax.axis_index("s")
    ebase = wid * E_PER_T
    cp = pltpu.async_copy(ei_hbm.at[1, pl.ds(ebase, E_PER_T)], idx_v, sem)

    zero = jnp.zeros((L,), jnp.float32)

    def zbody(i, carry):
        for k in range(_UNROLL):
            flags_v[pl.ds((i * _UNROLL + k) * L, L)] = zero
        return carry

    lax.fori_loop(0, N_PAD // (L * _UNROLL), zbody, 0)
    cp.wait()

    one = jnp.ones((L,), jnp.float32)

    def sbody(i, carry):
        for k in range(_UNROLL):
            iv = idx_v[pl.ds((i * _UNROLL + k) * L, L)]
            plsc.store_scatter(flags_v, [iv], one)
        return carry

    lax.fori_loop(0, E_PER_T // (L * _UNROLL), sbody, 0)
    pltpu.sync_copy(flags_v, part_hbm.at[wid])


B_TC = 5120


def _reduce_broadcast_tc(part_ref, out_ref):
    p = part_ref[...]
    ones = jnp.ones((NW, 1), jnp.float32)
    col = lax.dot_general(
        p, ones, (((0,), (0,)), ((), ())), preferred_element_type=jnp.float32
    )
    flag = jnp.where(col > 0.0, 1.0, 0.0)
    out_ref[...] = jnp.broadcast_to(flag, (B_TC, D_FEAT))


_reduce_broadcast = pl.pallas_call(
    _reduce_broadcast_tc,
    grid=(N_PAD // B_TC,),
    in_specs=[pl.BlockSpec((NW, B_TC), lambda i: (0, i))],
    out_specs=pl.BlockSpec((B_TC, D_FEAT), lambda i: (i, 0)),
    out_shape=jax.ShapeDtypeStruct((N_NODES, D_FEAT), jnp.float32),
)


def kernel(source_node_representation_with_coefficient, edge_index):
    del source_node_representation_with_coefficient
    part = _membership_scatter(edge_index)
    return _reduce_broadcast(part)


# speedup vs baseline: 13.3679x; 1.0638x over previous
import functools

import jax
import jax.numpy as jnp
from jax import lax
from jax.experimental import pallas as pl
from jax.experimental.pallas import tpu as pltpu
from jax.experimental.pallas import tpu_sc as plsc

N_NODES = 10000
N_EDGES = 320000
D_FEAT = 128

NC = 2
NS = 16
L = 16
NW = NC * NS
E_CH = 10240
E_LAST = N_EDGES - (NW - 1) * E_CH
N_PAD = 10240

_mesh = plsc.VectorSubcoreMesh(
    core_axis_name="c", subcore_axis_name="s", num_cores=NC, num_subcores=NS
)

_params = pltpu.CompilerParams(needs_layout_passes=False)

_UNROLL = 5


@functools.partial(
    pl.kernel,
    out_type=jax.ShapeDtypeStruct((NW * N_PAD,), jnp.float32),
    mesh=_mesh,
    scratch_types=[
        pltpu.VMEM((2, E_CH), jnp.int32),
        pltpu.VMEM((N_PAD,), jnp.float32),
        pltpu.SemaphoreType.DMA,
    ],
    compiler_params=_params,
)
def _membership_scatter(ei_hbm, part_hbm, idx_v, flags_v, sem):
    wid = lax.axis_index("c") * NS + lax.axis_index("s")
    ebase = wid * E_CH

    @pl.when(wid < NW - 1)
    def _():
        pltpu.async_copy(ei_hbm.at[:, pl.ds(ebase, E_CH)], idx_v, sem)

    @pl.when(wid == NW - 1)
    def _():
        pltpu.async_copy(
            ei_hbm.at[:, pl.ds(ebase, E_LAST)],
            idx_v.at[:, pl.ds(0, E_LAST)],
            sem,
        )

    zero = jnp.zeros((L,), jnp.float32)

    def zbody(i, carry):
        for k in range(_UNROLL):
            flags_v[pl.ds((i * _UNROLL + k) * L, L)] = zero
        return carry

    lax.fori_loop(0, N_PAD // (L * _UNROLL), zbody, 0)

    @pl.when(wid < NW - 1)
    def _():
        pltpu.make_async_copy(
            ei_hbm.at[:, pl.ds(ebase, E_CH)], idx_v, sem
        ).wait()

    @pl.when(wid == NW - 1)
    def _():
        pltpu.make_async_copy(
            ei_hbm.at[:, pl.ds(ebase, E_LAST)],
            idx_v.at[:, pl.ds(0, E_LAST)],
            sem,
        ).wait()

    one = jnp.ones((L,), jnp.float32)

    def sbody(i, carry):
        for k in range(_UNROLL):
            iv = idx_v[1, pl.ds((i * _UNROLL + k) * L, L)]
            plsc.store_scatter(flags_v, [iv], one)
        return carry

    n_edges_t = jnp.where(wid == NW - 1, E_LAST, E_CH)
    lax.fori_loop(0, n_edges_t // (L * _UNROLL), sbody, 0)
    pltpu.sync_copy(flags_v, part_hbm.at[pl.ds(wid * N_PAD, N_PAD)])


B_TC = 5120


def _reduce_broadcast_tc(part_ref, out_ref):
    p = part_ref[...]
    ones = jnp.ones((NW, 1), jnp.float32)
    col = lax.dot_general(
        p, ones, (((0,), (0,)), ((), ())), preferred_element_type=jnp.float32
    )
    flag = jnp.where(col > 0.0, 1.0, 0.0)
    out_ref[...] = jnp.broadcast_to(flag, (B_TC, D_FEAT))


_reduce_broadcast = pl.pallas_call(
    _reduce_broadcast_tc,
    grid=(N_PAD // B_TC,),
    in_specs=[pl.BlockSpec((NW, B_TC), lambda i: (0, i))],
    out_specs=pl.BlockSpec((B_TC, D_FEAT), lambda i: (i, 0)),
    out_shape=jax.ShapeDtypeStruct((N_NODES, D_FEAT), jnp.float32),
)


def kernel(source_node_representation_with_coefficient, edge_index):
    del source_node_representation_with_coefficient
    part = _membership_scatter(edge_index)
    return _reduce_broadcast(part.reshape(NW, N_PAD))
